# Initial kernel scaffold; baseline (speedup 1.0000x reference)
import jax
import jax.numpy as jnp
from jax.experimental import pallas as pl


def kernel(s, v, p, edge_d, edge_a, edge_r_norm, edge_e, edge_attr_initial_ohe, edge_attr_global_embedding, Wm1, bm1, Wm2, bm2, Wu1, bu1, Wu2, bu2, ln_g, ln_b, out_g, out_b, batch, edge_index):
    raise NotImplementedError("write your pallas kernel here")


# speedup vs baseline: 7.1981x; 7.1981x over previous
import functools
import math

import jax
import jax.numpy as jnp
from jax.experimental import pallas as pl

S = 64
V = 16
F = 16
G = 32
CUTOFF = 5.0
BLK = 3200


def _edge_kernel(ssrc_ref, stgt_ref, e_ref, oge_ref, d_ref, a_ref, rn_ref,
                 vsrc_ref, wm1_ref, bm1_ref, wm2_ref, bm2_ref,
                 scat_ref, enew_ref, *, has_v_in):
    ssrc = ssrc_ref[...]
    stgt = stgt_ref[...]
    e_in = e_ref[...] + oge_ref[...]
    d = d_ref[...]
    a = a_ref[...]
    rn = rn_ref[...]

    w1 = wm1_ref[...]
    pre = (jnp.dot(ssrc, w1[0:S, :], preferred_element_type=jnp.float32)
           + jnp.dot(stgt, w1[S:2 * S, :], preferred_element_type=jnp.float32)
           + jnp.dot(e_in, w1[2 * S:2 * S + F, :], preferred_element_type=jnp.float32)
           + d * w1[2 * S + F:2 * S + F + 1, :]
           + a * w1[2 * S + F + 1:2 * S + F + 2, :]
           + bm1_ref[...])
    h = pre * jax.nn.sigmoid(pre)
    o = jnp.dot(h, wm2_ref[...], preferred_element_type=jnp.float32) + bm2_ref[...]

    w = 0.5 * (jnp.cos((math.pi / CUTOFF) * d) + 1.0) * (d < CUTOFF).astype(jnp.float32)

    ms = o[:, 0:S] * w
    mvg = o[:, S:S + V]
    enew_ref[...] = o[:, S + V:S + V + F] * w
    pc = o[:, S + V + F:S + V + F + 1]

    mv = jnp.concatenate([rn[:, k:k + 1] * mvg for k in range(3)], axis=1)
    if has_v_in:
        mv = mv + vsrc_ref[...]
    mv = mv * w

    scat_ref[:, 0:S] = ms
    scat_ref[:, S:S + 48] = mv
    scat_ref[:, S + 48:S + 51] = pc * rn
    scat_ref[:, S + 51:] = jnp.zeros_like(scat_ref[:, S + 51:])


def _edge_pass(ssrc, stgt, e, oge, d, a, rn, vsrc, wm1, bm1, wm2p, bm2p, has_v_in):
    E = ssrc.shape[0]
    grid = (E // BLK,)
    blk = lambda c: pl.BlockSpec((BLK, c), lambda i: (i, 0))
    full = lambda r, c: pl.BlockSpec((r, c), lambda i: (0, 0))
    scat, enew = pl.pallas_call(
        functools.partial(_edge_kernel, has_v_in=has_v_in),
        grid=grid,
        in_specs=[blk(S), blk(S), blk(F), blk(F), blk(1), blk(1), blk(3),
                  blk(48), full(2 * S + F + 2, S), full(1, S),
                  full(S, 128), full(1, 128)],
        out_specs=[blk(128), blk(F)],
        out_shape=[jax.ShapeDtypeStruct((E, 128), jnp.float32),
                   jax.ShapeDtypeStruct((E, F), jnp.float32)],
    )(ssrc, stgt, e, oge, d, a, rn, vsrc, wm1, bm1, wm2p, bm2p)
    return scat, enew


def _ln(x, g, b):
    mu = x.mean(-1, keepdims=True)
    var = x.var(-1, keepdims=True)
    return (x - mu) / jnp.sqrt(var + 1e-6) * g + b


def _norm_sv(s, v, batch, g, b):
    s = _ln(s, g, b)
    n2 = jnp.sum(v * v, axis=(1, 2))
    sm = jax.ops.segment_sum(n2, batch, G)
    c = jax.ops.segment_sum(jnp.ones((n2.shape[0],), jnp.float32), batch, G)
    m = sm / jnp.clip(c, 1.0)
    denom = jnp.sqrt(m / (3.0 * V) + 1e-6)
    v = v / denom[batch][:, None, None]
    return s, v


def _edge_attrs(pos, src, tgt):
    r = pos[tgt] - pos[src]
    pn = jnp.linalg.norm(pos, axis=1, keepdims=True)
    pos_n = jnp.where(pn != 0.0, pos / jnp.where(pn == 0.0, 1.0, pn), 0.0)
    a = jnp.sum(pos_n[tgt] * pos_n[src], axis=-1)
    d = jnp.sqrt(jnp.clip(jnp.sum(r * r, axis=-1), 1e-6))
    r_norm = r / (1.0 + d[:, None])
    return d, a, r_norm


def kernel(s, v, p, edge_d, edge_a, edge_r_norm, edge_e, edge_attr_initial_ohe,
           edge_attr_global_embedding, Wm1, bm1, Wm2, bm2, Wu1, bu1, Wu2, bu2,
           ln_g, ln_b, out_g, out_b, batch, edge_index):
    N = s.shape[0]
    E = edge_d.shape[0]
    src, tgt = edge_index[0], edge_index[1]

    oge = edge_attr_initial_ohe + edge_attr_global_embedding
    inv_cnt = 1.0 / jnp.clip(
        jax.ops.segment_sum(jnp.ones((E,), jnp.float32), tgt, N), 1.0)

    def padw(wm2, bm2_):
        wp = jnp.zeros((S, 128), jnp.float32).at[:, :97].set(wm2)
        bp = jnp.zeros((1, 128), jnp.float32).at[:, :97].set(bm2_)
        return wp, bp

    d, a, rn = edge_d, edge_a, edge_r_norm
    e = edge_e
    zeros_vsrc = jnp.zeros((E, 48), jnp.float32)

    s0, v0 = _norm_sv(s, v, batch, ln_g[0], ln_b[0])
    wm2p, bm2p = padw(Wm2[0], bm2[0])
    scat, e = _edge_pass(s0[src], s0[tgt], e, oge, d[:, None], a[:, None], rn,
                         zeros_vsrc, Wm1[0], bm1[0][None], wm2p, bm2p, False)
    agg = jax.ops.segment_sum(scat, tgt, N) * inv_cnt[:, None]
    s_agg = agg[:, :S]
    v_agg = agg[:, S:S + 48].reshape(N, 3, V)
    p = p + agg[:, S + 48:S + 51]

    vn = jnp.sqrt(jnp.sum(v_agg * v_agg, axis=1) + 1e-6)
    ui = jnp.concatenate([s0, s_agg, vn], axis=-1)
    h2 = jax.nn.silu(ui @ Wu1[0] + bu1[0])
    o2 = h2 @ Wu2[0] + bu2[0]
    s = s0 + o2[:, :S]
    v = v0 + v_agg * o2[:, S:][:, None, :]

    d, a, rn = _edge_attrs(p, src, tgt)

    s1, v1 = _norm_sv(s, v, batch, ln_g[1], ln_b[1])
    wm2p, bm2p = padw(Wm2[1], bm2[1])
    scat, e = _edge_pass(s1[src], s1[tgt], e, oge, d[:, None], a[:, None], rn,
                         v1.reshape(N, 48)[src], Wm1[1], bm1[1][None], wm2p,
                         bm2p, True)
    agg = jax.ops.segment_sum(scat, tgt, N) * inv_cnt[:, None]
    s = s1 + agg[:, :S]
    v = v1 + agg[:, S:S + 48].reshape(N, 3, V)
    p = p + agg[:, S + 48:S + 51]

    s, v = _norm_sv(s, v, batch, out_g, out_b)
    return (s, v, e, p)


# speedup vs baseline: 13.7326x; 1.9078x over previous
import functools
import math

import jax
import jax.numpy as jnp
from jax import lax
from jax.experimental import pallas as pl
from jax.experimental.pallas import tpu as pltpu
from jax.experimental.pallas import tpu_sc as plsc

S = 64
V = 16
F = 16
G = 32
CUTOFF = 5.0
BLK = 2000
NW = 32


K_STREAMS = 8
PHASE = 4


def _sc_gather_call(table, idx2d):
    D = table.shape[1]
    ns = idx2d.shape[0]
    per_w = ns // NW
    iters = per_w // K_STREAMS
    mesh = plsc.VectorSubcoreMesh(core_axis_name="c", subcore_axis_name="s")

    @functools.partial(
        pl.kernel, mesh=mesh,
        out_type=jax.ShapeDtypeStruct((ns * 128, D), jnp.float32),
        scratch_types=[
            pltpu.VMEM((K_STREAMS, 128), jnp.int32),
            pltpu.VMEM((PHASE * 128, D), jnp.float32),
            pltpu.SemaphoreType.DMA,
        ],
    )
    def k(table_h, idx_h, out_h, idx_v, rows_v, sem):
        wid = lax.axis_index("s") * 2 + lax.axis_index("c")
        base = wid * per_w

        def body(t, carry):
            sb = base + t * K_STREAMS
            pltpu.sync_copy(idx_h.at[pl.ds(sb, K_STREAMS)], idx_v)
            for ph in range(K_STREAMS // PHASE):
                cops = [
                    pltpu.async_copy(table_h.at[idx_v.at[ph * PHASE + j]],
                                     rows_v.at[pl.ds(j * 128, 128)], sem)
                    for j in range(PHASE)
                ]
                for c in cops:
                    c.wait()
                pltpu.sync_copy(
                    rows_v, out_h.at[pl.ds((sb + ph * PHASE) * 128, PHASE * 128)])
            return carry

        lax.fori_loop(0, iters, body, 0)

    return k(table, idx2d)


def _edge_kernel(gsrc_ref, gtgt_ref, e_ref, oge_ref, d_ref, a_ref, rn_ref,
                 wm1_ref, bm1_ref, wm2_ref, bm2_ref, scat_ref, enew_ref,
                 *, layer1):
    gs = gsrc_ref[...]
    gt = gtgt_ref[...]
    ssrc = gs[:, 0:S]
    stgt = gt[:, 0:S]
    if layer1:
        r = gt[:, 112:115] - gs[:, 112:115]
        a = jnp.sum(gt[:, 115:118] * gs[:, 115:118], axis=1, keepdims=True)
        d = jnp.sqrt(jnp.clip(jnp.sum(r * r, axis=1, keepdims=True), 1e-6))
        rn = r / (1.0 + d)
    else:
        d = d_ref[...]
        a = a_ref[...]
        rn = rn_ref[...]

    e_in = e_ref[...] + oge_ref[...]

    w1 = wm1_ref[...]
    pre = (jnp.dot(ssrc, w1[0:S, :], preferred_element_type=jnp.float32)
           + jnp.dot(stgt, w1[S:2 * S, :], preferred_element_type=jnp.float32)
           + jnp.dot(e_in, w1[2 * S:2 * S + F, :], preferred_element_type=jnp.float32)
           + d * w1[2 * S + F:2 * S + F + 1, :]
           + a * w1[2 * S + F + 1:2 * S + F + 2, :]
           + bm1_ref[...])
    h = pre * jax.nn.sigmoid(pre)
    o = jnp.dot(h, wm2_ref[...], preferred_element_type=jnp.float32) + bm2_ref[...]

    w = 0.5 * (jnp.cos((math.pi / CUTOFF) * d) + 1.0) * (d < CUTOFF).astype(jnp.float32)

    ms = o[:, 0:S] * w
    mvg = o[:, S:S + V]
    enew_ref[...] = o[:, S + V:S + V + F] * w
    pc = o[:, S + V + F:S + V + F + 1]

    mv = jnp.concatenate([rn[:, k:k + 1] * mvg for k in range(3)], axis=1)
    if layer1:
        mv = mv + gs[:, S:S + 48]
    mv = mv * w

    scat_ref[:, 0:S] = ms
    scat_ref[:, S:S + 48] = mv
    scat_ref[:, S + 48:S + 51] = pc * rn
    scat_ref[:, S + 51:] = jnp.zeros_like(scat_ref[:, S + 51:])


def _edge_pass(g2, e, oge, d, a, rn, wm1, bm1, wm2p, bm2p, layer1):
    E = e.shape[0]
    nblk = E // BLK
    blk = lambda c: pl.BlockSpec((BLK, c), lambda i: (i, 0))
    full = lambda r, c: pl.BlockSpec((r, c), lambda i: (0, 0))
    scat, enew = pl.pallas_call(
        functools.partial(_edge_kernel, layer1=layer1),
        grid=(nblk,),
        in_specs=[pl.BlockSpec((BLK, 128), lambda i: (i, 0)),
                  pl.BlockSpec((BLK, 128), lambda i, n=nblk: (i + n, 0)),
                  blk(F), blk(F), blk(1), blk(1), blk(3),
                  full(2 * S + F + 2, S), full(1, S), full(S, 128),
                  full(1, 128)],
        out_specs=[blk(128), blk(F)],
        out_shape=[jax.ShapeDtypeStruct((E, 128), jnp.float32),
                   jax.ShapeDtypeStruct((E, F), jnp.float32)],
    )(g2, g2, e, oge, d, a, rn, wm1, bm1, wm2p, bm2p)
    return scat, enew


def _ln(x, g, b):
    mu = x.mean(-1, keepdims=True)
    var = x.var(-1, keepdims=True)
    return (x - mu) / jnp.sqrt(var + 1e-6) * g + b


def _norm_sv(s, v, batch, g, b):
    s = _ln(s, g, b)
    n2 = jnp.sum(v * v, axis=(1, 2))
    sm = jax.ops.segment_sum(n2, batch, G)
    c = jax.ops.segment_sum(jnp.ones((n2.shape[0],), jnp.float32), batch, G)
    m = sm / jnp.clip(c, 1.0)
    denom = jnp.sqrt(m / (3.0 * V) + 1e-6)
    v = v / denom[batch][:, None, None]
    return s, v


def kernel(s, v, p, edge_d, edge_a, edge_r_norm, edge_e, edge_attr_initial_ohe,
           edge_attr_global_embedding, Wm1, bm1, Wm2, bm2, Wu1, bu1, Wu2, bu2,
           ln_g, ln_b, out_g, out_b, batch, edge_index):
    N = s.shape[0]
    E = edge_d.shape[0]
    src, tgt = edge_index[0], edge_index[1]
    quant = 128 * NW * K_STREAMS
    mpad = ((2 * E + quant - 1) // quant) * quant
    idx2d = jnp.concatenate(
        [src, tgt, jnp.zeros((mpad - 2 * E,), src.dtype)]).reshape(-1, 128)

    oge = edge_attr_initial_ohe + edge_attr_global_embedding
    inv_cnt = 1.0 / jnp.clip(
        jax.ops.segment_sum(jnp.ones((E,), jnp.float32), tgt, N), 1.0)

    def padw(wm2, bm2_):
        wp = jnp.zeros((S, 128), jnp.float32).at[:, :97].set(wm2)
        bp = jnp.zeros((1, 128), jnp.float32).at[:, :97].set(bm2_)
        return wp, bp

    e = edge_e

    s0, v0 = _norm_sv(s, v, batch, ln_g[0], ln_b[0])
    t0 = jnp.concatenate([s0, jnp.zeros((N, 64), jnp.float32)], axis=1)
    g2 = _sc_gather_call(t0, idx2d)
    wm2p, bm2p = padw(Wm2[0], bm2[0])
    scat, e = _edge_pass(g2, e, oge, edge_d[:, None], edge_a[:, None],
                         edge_r_norm, Wm1[0], bm1[0][None], wm2p, bm2p, False)
    agg = jax.ops.segment_sum(scat, tgt, N) * inv_cnt[:, None]
    s_agg = agg[:, :S]
    v_agg = agg[:, S:S + 48].reshape(N, 3, V)
    p = p + agg[:, S + 48:S + 51]

    vn = jnp.sqrt(jnp.sum(v_agg * v_agg, axis=1) + 1e-6)
    ui = jnp.concatenate([s0, s_agg, vn], axis=-1)
    h2 = jax.nn.silu(ui @ Wu1[0] + bu1[0])
    o2 = h2 @ Wu2[0] + bu2[0]
    s = s0 + o2[:, :S]
    v = v0 + v_agg * o2[:, S:][:, None, :]

    pnorm = jnp.sqrt(jnp.sum(p * p, axis=1, keepdims=True))
    pos_n = jnp.where(pnorm != 0.0, p / jnp.where(pnorm == 0.0, 1.0, pnorm), 0.0)
    s1, v1 = _norm_sv(s, v, batch, ln_g[1], ln_b[1])
    t1 = jnp.concatenate([s1, v1.reshape(N, 48), p, pos_n,
                          jnp.zeros((N, 10), jnp.float32)], axis=1)
    g2 = _sc_gather_call(t1, idx2d)
    wm2p, bm2p = padw(Wm2[1], bm2[1])
    z1 = jnp.zeros((E, 1), jnp.float32)
    scat, e = _edge_pass(g2, e, oge, z1, z1, jnp.zeros((E, 3), jnp.float32),
                         Wm1[1], bm1[1][None], wm2p, bm2p, True)
    agg = jax.ops.segment_sum(scat, tgt, N) * inv_cnt[:, None]
    s = s1 + agg[:, :S]
    v = v1 + agg[:, S:S + 48].reshape(N, 3, V)
    p = p + agg[:, S + 48:S + 51]

    s, v = _norm_sv(s, v, batch, out_g, out_b)
    return (s, v, e, p)


# speedup vs baseline: 16.1204x; 1.1739x over previous
import functools
import math

import jax
import jax.numpy as jnp
from jax import lax
from jax.experimental import pallas as pl
from jax.experimental.pallas import tpu as pltpu
from jax.experimental.pallas import tpu_sc as plsc

S = 64
V = 16
F = 16
G = 32
CUTOFF = 5.0
BLK = 1280
NW = 32


K_STREAMS = 8
PHASE = 4


def _sc_gather_call(table, idx2d):
    D = table.shape[1]
    ns = idx2d.shape[0]
    per_w = ns // NW
    iters = per_w // K_STREAMS
    mesh = plsc.VectorSubcoreMesh(core_axis_name="c", subcore_axis_name="s")

    @functools.partial(
        pl.kernel, mesh=mesh,
        out_type=jax.ShapeDtypeStruct((ns * 128, D), jnp.float32),
        scratch_types=[
            pltpu.VMEM((K_STREAMS, 128), jnp.int32),
            pltpu.VMEM((PHASE * 128, D), jnp.float32),
            pltpu.SemaphoreType.DMA,
        ],
    )
    def k(table_h, idx_h, out_h, idx_v, rows_v, sem):
        wid = lax.axis_index("s") * 2 + lax.axis_index("c")
        base = wid * per_w

        def body(t, carry):
            sb = base + t * K_STREAMS
            pltpu.sync_copy(idx_h.at[pl.ds(sb, K_STREAMS)], idx_v)
            for ph in range(K_STREAMS // PHASE):
                cops = [
                    pltpu.async_copy(table_h.at[idx_v.at[ph * PHASE + j]],
                                     rows_v.at[pl.ds(j * 128, 128)], sem)
                    for j in range(PHASE)
                ]
                for c in cops:
                    c.wait()
                pltpu.sync_copy(
                    rows_v, out_h.at[pl.ds((sb + ph * PHASE) * 128, PHASE * 128)])
            return carry

        lax.fori_loop(0, iters, body, 0)

    return k(table, idx2d)


def _cutoff(d):
    return 0.5 * (jnp.cos((math.pi / CUTOFF) * d) + 1.0) * (d < CUTOFF).astype(jnp.float32)


def _mlp_tail(pre, wm2, bm2, d, rn, vadd, scat_ref, valid):
    h = pre * jax.nn.sigmoid(pre)
    o = jnp.dot(h, wm2, preferred_element_type=jnp.float32) + bm2
    w = _cutoff(d) * valid

    mvg = o[:, S:S + V]
    mv = jnp.concatenate([rn[:, k:k + 1] * mvg for k in range(3)], axis=1)
    if vadd is not None:
        mv = mv + vadd

    scat_ref[:, 0:S] = o[:, 0:S] * w
    scat_ref[:, S:S + 48] = mv * w
    scat_ref[:, S + 48:S + 51] = o[:, S + V + F:S + V + F + 1] * rn
    scat_ref[:, S + 51:] = jnp.zeros_like(scat_ref[:, S + 51:])
    return o[:, S + V:S + V + F] * w


def _edge_kernel0(gsrc_ref, gtgt_ref, et_ref, ogt_ref, edgt_ref,
                  wm1a_ref, wm1x_ref, bm1_ref, wm2_ref, bm2_ref,
                  scat_ref, enewt_ref):
    x = jnp.concatenate([et_ref[...] + ogt_ref[...], edgt_ref[...]], axis=0)
    w1a = wm1a_ref[...]
    pre = (jnp.dot(gsrc_ref[...][:, 0:S], w1a[0:S, :], preferred_element_type=jnp.float32)
           + jnp.dot(gtgt_ref[...][:, 0:S], w1a[S:2 * S, :], preferred_element_type=jnp.float32)
           + lax.dot_general(x, wm1x_ref[...], (((0,), (0,)), ((), ())),
                             preferred_element_type=jnp.float32)
           + bm1_ref[...])
    edgr = jnp.transpose(edgt_ref[...])
    d = edgr[:, 0:1]
    rn = edgr[:, 2:5]
    enew = _mlp_tail(pre, wm2_ref[...], bm2_ref[...], d, rn, None,
                     scat_ref, 1.0)
    enewt_ref[...] = jnp.transpose(enew)


def _edge_kernel1(gsrc_ref, gtgt_ref, et_ref, ogt_ref,
                  wm1a_ref, wm1x_ref, bm1_ref, wm2_ref, bm2_ref,
                  scat_ref, enew_ref):
    gs = gsrc_ref[...]
    gt = gtgt_ref[...]
    r = gt[:, 112:115] - gs[:, 112:115]
    a = jnp.sum(gt[:, 115:118] * gs[:, 115:118], axis=1, keepdims=True)
    d = jnp.sqrt(jnp.clip(jnp.sum(r * r, axis=1, keepdims=True), 1e-6))
    rn = r / (1.0 + d)

    x = et_ref[...] + ogt_ref[...]
    w1a = wm1a_ref[...]
    w1x = wm1x_ref[...]
    pre = (jnp.dot(gs[:, 0:S], w1a[0:S, :], preferred_element_type=jnp.float32)
           + jnp.dot(gt[:, 0:S], w1a[S:2 * S, :], preferred_element_type=jnp.float32)
           + lax.dot_general(x, w1x[0:F, :], (((0,), (0,)), ((), ())),
                             preferred_element_type=jnp.float32)
           + d * w1x[F:F + 1, :]
           + a * w1x[F + 1:F + 2, :]
           + bm1_ref[...])
    enew_ref[...] = _mlp_tail(pre, wm2_ref[...], bm2_ref[...], d, rn,
                              gs[:, S:S + 48], scat_ref, 1.0)


def _edge_pass(g2, et, ogt, edgt, wm1a, wm1x, bm1, wm2p, bm2p, layer1, E):
    nblk = E // BLK
    colt = lambda r: pl.BlockSpec((r, BLK), lambda i: (0, i))
    full = lambda r, c: pl.BlockSpec((r, c), lambda i: (0, 0))
    gspec_s = pl.BlockSpec((BLK, 128), lambda i: (i, 0))
    gspec_t = pl.BlockSpec((BLK, 128), lambda i, n=nblk: (i + n, 0))
    if layer1:
        kern = _edge_kernel1
        in_specs = [gspec_s, gspec_t, colt(F), colt(F)]
        ops = (g2, g2, et, ogt)
        out_specs = [pl.BlockSpec((BLK, 128), lambda i: (i, 0)),
                     pl.BlockSpec((BLK, F), lambda i: (i, 0))]
        out_shape = [jax.ShapeDtypeStruct((E, 128), jnp.float32),
                     jax.ShapeDtypeStruct((E, F), jnp.float32)]
    else:
        kern = _edge_kernel0
        in_specs = [gspec_s, gspec_t, colt(F), colt(F), colt(8)]
        ops = (g2, g2, et, ogt, edgt)
        out_specs = [pl.BlockSpec((BLK, 128), lambda i: (i, 0)),
                     pl.BlockSpec((F, BLK), lambda i: (0, i))]
        out_shape = [jax.ShapeDtypeStruct((E, 128), jnp.float32),
                     jax.ShapeDtypeStruct((F, E), jnp.float32)]
    in_specs += [full(128, S), full(24, S), full(1, S), full(S, 128),
                 full(1, 128)]
    scat, enew = pl.pallas_call(
        kern,
        grid=(nblk,),
        in_specs=in_specs,
        out_specs=out_specs,
        out_shape=out_shape,
    )(*ops, wm1a, wm1x, bm1, wm2p, bm2p)
    return scat, enew


def _ln(x, g, b):
    mu = x.mean(-1, keepdims=True)
    var = x.var(-1, keepdims=True)
    return (x - mu) / jnp.sqrt(var + 1e-6) * g + b


def _norm_sv(s, vf, batch, g, b):
    s = _ln(s, g, b)
    n2 = jnp.sum(vf * vf, axis=1)
    sm = jax.ops.segment_sum(n2, batch, G)
    c = jax.ops.segment_sum(jnp.ones((n2.shape[0],), jnp.float32), batch, G)
    m = sm / jnp.clip(c, 1.0)
    denom = jnp.sqrt(m / (3.0 * V) + 1e-6)
    vf = vf / denom[batch][:, None]
    return s, vf


def _split_weights(wm1, bm1_, wm2, bm2_):
    w1a = wm1[0:2 * S, :]
    w1x = jnp.concatenate([wm1[2 * S:2 * S + F + 2, :],
                           jnp.zeros((6, S), jnp.float32)], axis=0)
    wp = jnp.zeros((S, 128), jnp.float32).at[:, :97].set(wm2)
    bp = jnp.zeros((1, 128), jnp.float32).at[:, :97].set(bm2_)
    return w1a, w1x, bm1_[None], wp, bp


def kernel(s, v, p, edge_d, edge_a, edge_r_norm, edge_e, edge_attr_initial_ohe,
           edge_attr_global_embedding, Wm1, bm1, Wm2, bm2, Wu1, bu1, Wu2, bu2,
           ln_g, ln_b, out_g, out_b, batch, edge_index):
    N = s.shape[0]
    E = edge_d.shape[0]
    src, tgt = edge_index[0], edge_index[1]
    quant = 128 * NW * K_STREAMS
    mpad = ((2 * E + quant - 1) // quant) * quant
    idx2d = jnp.concatenate(
        [src, tgt, jnp.zeros((mpad - 2 * E,), src.dtype)]).reshape(-1, 128)

    ogt = jnp.transpose(edge_attr_initial_ohe + edge_attr_global_embedding)
    e0t = jnp.transpose(edge_e)
    zr = jnp.zeros((E,), jnp.float32)
    edgt = jnp.stack([edge_d, edge_a, edge_r_norm[:, 0], edge_r_norm[:, 1],
                      edge_r_norm[:, 2], zr, zr, zr], axis=0)

    inv_cnt = 1.0 / jnp.clip(
        jax.ops.segment_sum(jnp.ones((E,), jnp.float32), tgt, N), 1.0)

    vf = v.reshape(N, 3 * V)

    s0, v0 = _norm_sv(s, vf, batch, ln_g[0], ln_b[0])
    t0 = jnp.concatenate([s0, jnp.zeros((N, 64), jnp.float32)], axis=1)
    g2 = _sc_gather_call(t0, idx2d)
    wts = _split_weights(Wm1[0], bm1[0], Wm2[0], bm2[0])
    scat, e1t = _edge_pass(g2, e0t, ogt, edgt, *wts, False, E)
    agg = jax.ops.segment_sum(scat, tgt, N) * inv_cnt[:, None]
    s_agg = agg[:, :S]
    v_agg = agg[:, S:S + 48]
    p = p + agg[:, S + 48:S + 51]

    vn = jnp.sqrt(v_agg[:, 0:V] ** 2 + v_agg[:, V:2 * V] ** 2
                  + v_agg[:, 2 * V:3 * V] ** 2 + 1e-6)
    ui = jnp.concatenate([s0, s_agg, vn], axis=-1)
    h2 = jax.nn.silu(ui @ Wu1[0] + bu1[0])
    o2 = h2 @ Wu2[0] + bu2[0]
    s = s0 + o2[:, :S]
    vf = v0 + v_agg * jnp.tile(o2[:, S:], (1, 3))

    pnorm = jnp.sqrt(jnp.sum(p * p, axis=1, keepdims=True))
    pos_n = jnp.where(pnorm != 0.0, p / jnp.where(pnorm == 0.0, 1.0, pnorm), 0.0)
    s1, v1 = _norm_sv(s, vf, batch, ln_g[1], ln_b[1])
    t1 = jnp.concatenate([s1, v1, p, pos_n, jnp.zeros((N, 10), jnp.float32)],
                         axis=1)
    g2 = _sc_gather_call(t1, idx2d)
    wts = _split_weights(Wm1[1], bm1[1], Wm2[1], bm2[1])
    scat, e2 = _edge_pass(g2, e1t, ogt, None, *wts, True, E)
    agg = jax.ops.segment_sum(scat, tgt, N) * inv_cnt[:, None]
    s = s1 + agg[:, :S]
    vf = v1 + agg[:, S:S + 48]
    p = p + agg[:, S + 48:S + 51]

    s, vf = _norm_sv(s, vf, batch, out_g, out_b)
    return (s, vf.reshape(N, 3, V), e2, p)


# speedup vs baseline: 16.3191x; 1.0123x over previous
import functools
import math

import jax
import jax.numpy as jnp
from jax import lax
from jax.experimental import pallas as pl
from jax.experimental.pallas import tpu as pltpu
from jax.experimental.pallas import tpu_sc as plsc

S = 64
V = 16
F = 16
G = 32
CUTOFF = 5.0
BLK = 6400
NW = 32


K_STREAMS = 8
PHASE = 4


def _sc_gather_call(table, idx2d):
    D = table.shape[1]
    ns = idx2d.shape[0]
    per_w = ns // NW
    iters = per_w // K_STREAMS
    mesh = plsc.VectorSubcoreMesh(core_axis_name="c", subcore_axis_name="s")

    @functools.partial(
        pl.kernel, mesh=mesh,
        out_type=jax.ShapeDtypeStruct((ns * 128, D), jnp.float32),
        scratch_types=[
            pltpu.VMEM((K_STREAMS, 128), jnp.int32),
            pltpu.VMEM((PHASE * 128, D), jnp.float32),
            pltpu.SemaphoreType.DMA,
        ],
    )
    def k(table_h, idx_h, out_h, idx_v, rows_v, sem):
        wid = lax.axis_index("s") * 2 + lax.axis_index("c")
        base = wid * per_w

        def body(t, carry):
            sb = base + t * K_STREAMS
            pltpu.sync_copy(idx_h.at[pl.ds(sb, K_STREAMS)], idx_v)
            for ph in range(K_STREAMS // PHASE):
                cops = [
                    pltpu.async_copy(table_h.at[idx_v.at[ph * PHASE + j]],
                                     rows_v.at[pl.ds(j * 128, 128)], sem)
                    for j in range(PHASE)
                ]
                for c in cops:
                    c.wait()
                pltpu.sync_copy(
                    rows_v, out_h.at[pl.ds((sb + ph * PHASE) * 128, PHASE * 128)])
            return carry

        lax.fori_loop(0, iters, body, 0)

    return k(table, idx2d)


def _cutoff(d):
    return 0.5 * (jnp.cos((math.pi / CUTOFF) * d) + 1.0) * (d < CUTOFF).astype(jnp.float32)


def _mlp_tail(pre, wm2, bm2, d, rn, vadd, scat_ref, valid):
    h = pre * jax.nn.sigmoid(pre)
    o = jnp.dot(h, wm2, preferred_element_type=jnp.float32) + bm2
    w = _cutoff(d) * valid

    mvg = o[:, S:S + V]
    mv = jnp.concatenate([rn[:, k:k + 1] * mvg for k in range(3)], axis=1)
    if vadd is not None:
        mv = mv + vadd

    scat_ref[:, 0:S] = o[:, 0:S] * w
    scat_ref[:, S:S + 48] = mv * w
    scat_ref[:, S + 48:S + 51] = o[:, S + V + F:S + V + F + 1] * rn
    scat_ref[:, S + 51:] = jnp.zeros_like(scat_ref[:, S + 51:])
    return o[:, S + V:S + V + F] * w


def _edge_kernel0(gsrc_ref, gtgt_ref, et_ref, ogt_ref, edgt_ref,
                  wm1a_ref, wm1x_ref, bm1_ref, wm2_ref, bm2_ref,
                  scat_ref, enewt_ref):
    x = jnp.concatenate([et_ref[...] + ogt_ref[...], edgt_ref[...]], axis=0)
    w1a = wm1a_ref[...]
    pre = (jnp.dot(gsrc_ref[...][:, 0:S], w1a[0:S, :], preferred_element_type=jnp.float32)
           + jnp.dot(gtgt_ref[...][:, 0:S], w1a[S:2 * S, :], preferred_element_type=jnp.float32)
           + lax.dot_general(x, wm1x_ref[...], (((0,), (0,)), ((), ())),
                             preferred_element_type=jnp.float32)
           + bm1_ref[...])
    edgr = jnp.transpose(edgt_ref[...])
    d = edgr[:, 0:1]
    rn = edgr[:, 2:5]
    enew = _mlp_tail(pre, wm2_ref[...], bm2_ref[...], d, rn, None,
                     scat_ref, 1.0)
    enewt_ref[...] = jnp.transpose(enew)


def _edge_kernel1(gsrc_ref, gtgt_ref, et_ref, ogt_ref,
                  wm1a_ref, wm1x_ref, bm1_ref, wm2_ref, bm2_ref,
                  scat_ref, enew_ref):
    gs = gsrc_ref[...]
    gt = gtgt_ref[...]
    r = gt[:, 112:115] - gs[:, 112:115]
    a = jnp.sum(gt[:, 115:118] * gs[:, 115:118], axis=1, keepdims=True)
    d = jnp.sqrt(jnp.clip(jnp.sum(r * r, axis=1, keepdims=True), 1e-6))
    rn = r / (1.0 + d)

    x = et_ref[...] + ogt_ref[...]
    w1a = wm1a_ref[...]
    w1x = wm1x_ref[...]
    pre = (jnp.dot(gs[:, 0:S], w1a[0:S, :], preferred_element_type=jnp.float32)
           + jnp.dot(gt[:, 0:S], w1a[S:2 * S, :], preferred_element_type=jnp.float32)
           + lax.dot_general(x, w1x[0:F, :], (((0,), (0,)), ((), ())),
                             preferred_element_type=jnp.float32)
           + d * w1x[F:F + 1, :]
           + a * w1x[F + 1:F + 2, :]
           + bm1_ref[...])
    enew_ref[...] = _mlp_tail(pre, wm2_ref[...], bm2_ref[...], d, rn,
                              gs[:, S:S + 48], scat_ref, 1.0)


def _edge_pass(g2, et, ogt, edgt, wm1a, wm1x, bm1, wm2p, bm2p, layer1, E):
    nblk = E // BLK
    colt = lambda r: pl.BlockSpec((r, BLK), lambda i: (0, i))
    full = lambda r, c: pl.BlockSpec((r, c), lambda i: (0, 0))
    gspec_s = pl.BlockSpec((BLK, 128), lambda i: (i, 0))
    gspec_t = pl.BlockSpec((BLK, 128), lambda i, n=nblk: (i + n, 0))
    if layer1:
        kern = _edge_kernel1
        in_specs = [gspec_s, gspec_t, colt(F), colt(F)]
        ops = (g2, g2, et, ogt)
        out_specs = [pl.BlockSpec((BLK, 128), lambda i: (i, 0)),
                     pl.BlockSpec((BLK, F), lambda i: (i, 0))]
        out_shape = [jax.ShapeDtypeStruct((E, 128), jnp.float32),
                     jax.ShapeDtypeStruct((E, F), jnp.float32)]
    else:
        kern = _edge_kernel0
        in_specs = [gspec_s, gspec_t, colt(F), colt(F), colt(8)]
        ops = (g2, g2, et, ogt, edgt)
        out_specs = [pl.BlockSpec((BLK, 128), lambda i: (i, 0)),
                     pl.BlockSpec((F, BLK), lambda i: (0, i))]
        out_shape = [jax.ShapeDtypeStruct((E, 128), jnp.float32),
                     jax.ShapeDtypeStruct((F, E), jnp.float32)]
    in_specs += [full(128, S), full(24, S), full(1, S), full(S, 128),
                 full(1, 128)]
    scat, enew = pl.pallas_call(
        kern,
        grid=(nblk,),
        in_specs=in_specs,
        out_specs=out_specs,
        out_shape=out_shape,
    )(*ops, wm1a, wm1x, bm1, wm2p, bm2p)
    return scat, enew


def _ln(x, g, b):
    mu = x.mean(-1, keepdims=True)
    var = x.var(-1, keepdims=True)
    return (x - mu) / jnp.sqrt(var + 1e-6) * g + b


def _norm_sv(s, vf, batch, g, b):
    s = _ln(s, g, b)
    n2 = jnp.sum(vf * vf, axis=1)
    sm = jax.ops.segment_sum(n2, batch, G)
    c = jax.ops.segment_sum(jnp.ones((n2.shape[0],), jnp.float32), batch, G)
    m = sm / jnp.clip(c, 1.0)
    denom = jnp.sqrt(m / (3.0 * V) + 1e-6)
    vf = vf / denom[batch][:, None]
    return s, vf


def _split_weights(wm1, bm1_, wm2, bm2_):
    w1a = wm1[0:2 * S, :]
    w1x = jnp.concatenate([wm1[2 * S:2 * S + F + 2, :],
                           jnp.zeros((6, S), jnp.float32)], axis=0)
    wp = jnp.zeros((S, 128), jnp.float32).at[:, :97].set(wm2)
    bp = jnp.zeros((1, 128), jnp.float32).at[:, :97].set(bm2_)
    return w1a, w1x, bm1_[None], wp, bp


def kernel(s, v, p, edge_d, edge_a, edge_r_norm, edge_e, edge_attr_initial_ohe,
           edge_attr_global_embedding, Wm1, bm1, Wm2, bm2, Wu1, bu1, Wu2, bu2,
           ln_g, ln_b, out_g, out_b, batch, edge_index):
    N = s.shape[0]
    E = edge_d.shape[0]
    src, tgt = edge_index[0], edge_index[1]
    quant = 128 * NW * K_STREAMS
    mpad = ((2 * E + quant - 1) // quant) * quant
    idx2d = jnp.concatenate(
        [src, tgt, jnp.zeros((mpad - 2 * E,), src.dtype)]).reshape(-1, 128)

    ogt = jnp.transpose(edge_attr_initial_ohe + edge_attr_global_embedding)
    e0t = jnp.transpose(edge_e)
    zr = jnp.zeros((E,), jnp.float32)
    edgt = jnp.stack([edge_d, edge_a, edge_r_norm[:, 0], edge_r_norm[:, 1],
                      edge_r_norm[:, 2], zr, zr, zr], axis=0)

    inv_cnt = 1.0 / jnp.clip(
        jax.ops.segment_sum(jnp.ones((E,), jnp.float32), tgt, N), 1.0)

    vf = v.reshape(N, 3 * V)

    s0, v0 = _norm_sv(s, vf, batch, ln_g[0], ln_b[0])
    t0 = jnp.concatenate([s0, jnp.zeros((N, 64), jnp.float32)], axis=1)
    g2 = _sc_gather_call(t0, idx2d)
    wts = _split_weights(Wm1[0], bm1[0], Wm2[0], bm2[0])
    scat, e1t = _edge_pass(g2, e0t, ogt, edgt, *wts, False, E)
    agg = jax.ops.segment_sum(scat, tgt, N) * inv_cnt[:, None]
    s_agg = agg[:, :S]
    v_agg = agg[:, S:S + 48]
    p = p + agg[:, S + 48:S + 51]

    vn = jnp.sqrt(v_agg[:, 0:V] ** 2 + v_agg[:, V:2 * V] ** 2
                  + v_agg[:, 2 * V:3 * V] ** 2 + 1e-6)
    ui = jnp.concatenate([s0, s_agg, vn], axis=-1)
    h2 = jax.nn.silu(ui @ Wu1[0] + bu1[0])
    o2 = h2 @ Wu2[0] + bu2[0]
    s = s0 + o2[:, :S]
    vf = v0 + v_agg * jnp.tile(o2[:, S:], (1, 3))

    pnorm = jnp.sqrt(jnp.sum(p * p, axis=1, keepdims=True))
    pos_n = jnp.where(pnorm != 0.0, p / jnp.where(pnorm == 0.0, 1.0, pnorm), 0.0)
    s1, v1 = _norm_sv(s, vf, batch, ln_g[1], ln_b[1])
    t1 = jnp.concatenate([s1, v1, p, pos_n, jnp.zeros((N, 10), jnp.float32)],
                         axis=1)
    g2 = _sc_gather_call(t1, idx2d)
    wts = _split_weights(Wm1[1], bm1[1], Wm2[1], bm2[1])
    scat, e2 = _edge_pass(g2, e1t, ogt, None, *wts, True, E)
    agg = jax.ops.segment_sum(scat, tgt, N) * inv_cnt[:, None]
    s = s1 + agg[:, :S]
    vf = v1 + agg[:, S:S + 48]
    p = p + agg[:, S + 48:S + 51]

    s, vf = _norm_sv(s, vf, batch, out_g, out_b)
    return (s, vf.reshape(N, 3, V), e2, p)


# speedup vs baseline: 17.0552x; 1.0451x over previous
import functools
import math

import jax
import jax.numpy as jnp
from jax import lax
from jax.experimental import pallas as pl
from jax.experimental.pallas import tpu as pltpu
from jax.experimental.pallas import tpu_sc as plsc

S = 64
V = 16
F = 16
G = 32
CUTOFF = 5.0
BLK = 3200
NW = 32


K_STREAMS = 8
PHASE = 4


def _sc_gather_call(table, idx2d):
    D = table.shape[1]
    ns = idx2d.shape[0]
    per_w = ns // NW
    iters = per_w // K_STREAMS
    mesh = plsc.VectorSubcoreMesh(core_axis_name="c", subcore_axis_name="s")

    @functools.partial(
        pl.kernel, mesh=mesh,
        out_type=jax.ShapeDtypeStruct((ns * 128, D), jnp.float32),
        scratch_types=[
            pltpu.VMEM((K_STREAMS, 128), jnp.int32),
            pltpu.VMEM((PHASE * 128, D), jnp.float32),
            pltpu.SemaphoreType.DMA,
        ],
    )
    def k(table_h, idx_h, out_h, idx_v, rows_v, sem):
        wid = lax.axis_index("s") * 2 + lax.axis_index("c")
        base = wid * per_w

        def body(t, carry):
            sb = base + t * K_STREAMS
            pltpu.sync_copy(idx_h.at[pl.ds(sb, K_STREAMS)], idx_v)
            for ph in range(K_STREAMS // PHASE):
                cops = [
                    pltpu.async_copy(table_h.at[idx_v.at[ph * PHASE + j]],
                                     rows_v.at[pl.ds(j * 128, 128)], sem)
                    for j in range(PHASE)
                ]
                for c in cops:
                    c.wait()
                pltpu.sync_copy(
                    rows_v, out_h.at[pl.ds((sb + ph * PHASE) * 128, PHASE * 128)])
            return carry

        lax.fori_loop(0, iters, body, 0)

    return k(table, idx2d)


def _cutoff(d):
    return 0.5 * (jnp.cos((math.pi / CUTOFF) * d) + 1.0) * (d < CUTOFF).astype(jnp.float32)


def _mlp_tail(pre, wm2, bm2, d, rn, vadd, scat_ref, valid):
    h = pre * jax.nn.sigmoid(pre)
    o = jnp.dot(h, wm2, preferred_element_type=jnp.float32) + bm2
    w = _cutoff(d) * valid

    mvg = o[:, S:S + V]
    mv = jnp.concatenate([rn[:, k:k + 1] * mvg for k in range(3)], axis=1)
    if vadd is not None:
        mv = mv + vadd

    scat_ref[:, 0:S] = o[:, 0:S] * w
    scat_ref[:, S:S + 48] = mv * w
    scat_ref[:, S + 48:S + 51] = o[:, S + V + F:S + V + F + 1] * rn
    scat_ref[:, S + 51:] = jnp.zeros_like(scat_ref[:, S + 51:])
    return o[:, S + V:S + V + F] * w


def _edge_kernel0(gsrc_ref, gtgt_ref, et_ref, ogt_ref, edgt_ref,
                  wm1a_ref, wm1x_ref, bm1_ref, wm2_ref, bm2_ref,
                  scat_ref, enewt_ref):
    x = jnp.concatenate([et_ref[...] + ogt_ref[...], edgt_ref[...]], axis=0)
    w1a = wm1a_ref[...]
    pre = (jnp.dot(gsrc_ref[...][:, 0:S], w1a[0:S, :], preferred_element_type=jnp.float32)
           + jnp.dot(gtgt_ref[...][:, 0:S], w1a[S:2 * S, :], preferred_element_type=jnp.float32)
           + lax.dot_general(x, wm1x_ref[...], (((0,), (0,)), ((), ())),
                             preferred_element_type=jnp.float32)
           + bm1_ref[...])
    edgr = jnp.transpose(edgt_ref[...])
    d = edgr[:, 0:1]
    rn = edgr[:, 2:5]
    enew = _mlp_tail(pre, wm2_ref[...], bm2_ref[...], d, rn, None,
                     scat_ref, 1.0)
    enewt_ref[...] = jnp.transpose(enew)


def _edge_kernel1(gsrc_ref, gtgt_ref, et_ref, ogt_ref,
                  wm1a_ref, wm1x_ref, bm1_ref, wm2_ref, bm2_ref,
                  scat_ref, enew_ref):
    gs = gsrc_ref[...]
    gt = gtgt_ref[...]
    r = gt[:, 112:115] - gs[:, 112:115]
    a = jnp.sum(gt[:, 115:118] * gs[:, 115:118], axis=1, keepdims=True)
    d = jnp.sqrt(jnp.clip(jnp.sum(r * r, axis=1, keepdims=True), 1e-6))
    rn = r / (1.0 + d)

    x = et_ref[...] + ogt_ref[...]
    w1a = wm1a_ref[...]
    w1x = wm1x_ref[...]
    pre = (jnp.dot(gs[:, 0:S], w1a[0:S, :], preferred_element_type=jnp.float32)
           + jnp.dot(gt[:, 0:S], w1a[S:2 * S, :], preferred_element_type=jnp.float32)
           + lax.dot_general(x, w1x[0:F, :], (((0,), (0,)), ((), ())),
                             preferred_element_type=jnp.float32)
           + d * w1x[F:F + 1, :]
           + a * w1x[F + 1:F + 2, :]
           + bm1_ref[...])
    enew_ref[...] = _mlp_tail(pre, wm2_ref[...], bm2_ref[...], d, rn,
                              gs[:, S:S + 48], scat_ref, 1.0)


def _edge_pass(g2, et, ogt, edgt, wm1a, wm1x, bm1, wm2p, bm2p, layer1, E):
    nblk = E // BLK
    colt = lambda r: pl.BlockSpec((r, BLK), lambda i: (0, i))
    full = lambda r, c: pl.BlockSpec((r, c), lambda i: (0, 0))
    gspec_s = pl.BlockSpec((BLK, 128), lambda i: (i, 0))
    gspec_t = pl.BlockSpec((BLK, 128), lambda i, n=nblk: (i + n, 0))
    if layer1:
        kern = _edge_kernel1
        in_specs = [gspec_s, gspec_t, colt(F), colt(F)]
        ops = (g2, g2, et, ogt)
        out_specs = [pl.BlockSpec((BLK, 128), lambda i: (i, 0)),
                     pl.BlockSpec((BLK, F), lambda i: (i, 0))]
        out_shape = [jax.ShapeDtypeStruct((E, 128), jnp.float32),
                     jax.ShapeDtypeStruct((E, F), jnp.float32)]
    else:
        kern = _edge_kernel0
        in_specs = [gspec_s, gspec_t, colt(F), colt(F), colt(8)]
        ops = (g2, g2, et, ogt, edgt)
        out_specs = [pl.BlockSpec((BLK, 128), lambda i: (i, 0)),
                     pl.BlockSpec((F, BLK), lambda i: (0, i))]
        out_shape = [jax.ShapeDtypeStruct((E, 128), jnp.float32),
                     jax.ShapeDtypeStruct((F, E), jnp.float32)]
    in_specs += [full(128, S), full(24, S), full(1, S), full(S, 128),
                 full(1, 128)]
    scat, enew = pl.pallas_call(
        kern,
        grid=(nblk,),
        in_specs=in_specs,
        out_specs=out_specs,
        out_shape=out_shape,
    )(*ops, wm1a, wm1x, bm1, wm2p, bm2p)
    return scat, enew


def _ln(x, g, b):
    mu = x.mean(-1, keepdims=True)
    var = x.var(-1, keepdims=True)
    return (x - mu) / jnp.sqrt(var + 1e-6) * g + b


def _norm_sv(s, vf, batch, g, b):
    s = _ln(s, g, b)
    n2 = jnp.sum(vf * vf, axis=1)
    sm = jax.ops.segment_sum(n2, batch, G)
    c = jax.ops.segment_sum(jnp.ones((n2.shape[0],), jnp.float32), batch, G)
    m = sm / jnp.clip(c, 1.0)
    denom = jnp.sqrt(m / (3.0 * V) + 1e-6)
    vf = vf / denom[batch][:, None]
    return s, vf


def _split_weights(wm1, bm1_, wm2, bm2_):
    w1a = wm1[0:2 * S, :]
    w1x = jnp.concatenate([wm1[2 * S:2 * S + F + 2, :],
                           jnp.zeros((6, S), jnp.float32)], axis=0)
    wp = jnp.zeros((S, 128), jnp.float32).at[:, :97].set(wm2)
    bp = jnp.zeros((1, 128), jnp.float32).at[:, :97].set(bm2_)
    return w1a, w1x, bm1_[None], wp, bp


def kernel(s, v, p, edge_d, edge_a, edge_r_norm, edge_e, edge_attr_initial_ohe,
           edge_attr_global_embedding, Wm1, bm1, Wm2, bm2, Wu1, bu1, Wu2, bu2,
           ln_g, ln_b, out_g, out_b, batch, edge_index):
    N = s.shape[0]
    E = edge_d.shape[0]
    src, tgt = edge_index[0], edge_index[1]
    EH = E // 2
    halves = [(0, EH), (EH, E)]
    quant = 128 * NW * K_STREAMS

    def half_idx(lo, hi):
        n = 2 * (hi - lo)
        mp = ((n + quant - 1) // quant) * quant
        return jnp.concatenate(
            [src[lo:hi], tgt[lo:hi],
             jnp.zeros((mp - n,), src.dtype)]).reshape(-1, 128)

    idxh = [half_idx(lo, hi) for lo, hi in halves]

    ogt = jnp.transpose(edge_attr_initial_ohe + edge_attr_global_embedding)
    e0t = jnp.transpose(edge_e)
    zr = jnp.zeros((E,), jnp.float32)
    edgt = jnp.stack([edge_d, edge_a, edge_r_norm[:, 0], edge_r_norm[:, 1],
                      edge_r_norm[:, 2], zr, zr, zr], axis=0)

    inv_cnt = 1.0 / jnp.clip(
        jax.ops.segment_sum(jnp.ones((E,), jnp.float32), tgt, N), 1.0)

    vf = v.reshape(N, 3 * V)

    s0, v0 = _norm_sv(s, vf, batch, ln_g[0], ln_b[0])
    t0 = jnp.concatenate([s0, jnp.zeros((N, 64), jnp.float32)], axis=1)
    wts = _split_weights(Wm1[0], bm1[0], Wm2[0], bm2[0])
    aggs = []
    e1ts = []
    for h, (lo, hi) in enumerate(halves):
        g2 = _sc_gather_call(t0, idxh[h])
        scat, e1t_h = _edge_pass(g2, e0t[:, lo:hi], ogt[:, lo:hi],
                                 edgt[:, lo:hi], *wts, False, hi - lo)
        aggs.append(jax.ops.segment_sum(scat, tgt[lo:hi], N))
        e1ts.append(e1t_h)
    agg = (aggs[0] + aggs[1]) * inv_cnt[:, None]
    s_agg = agg[:, :S]
    v_agg = agg[:, S:S + 48]
    p = p + agg[:, S + 48:S + 51]

    vn = jnp.sqrt(v_agg[:, 0:V] ** 2 + v_agg[:, V:2 * V] ** 2
                  + v_agg[:, 2 * V:3 * V] ** 2 + 1e-6)
    ui = jnp.concatenate([s0, s_agg, vn], axis=-1)
    h2 = jax.nn.silu(ui @ Wu1[0] + bu1[0])
    o2 = h2 @ Wu2[0] + bu2[0]
    s = s0 + o2[:, :S]
    vf = v0 + v_agg * jnp.tile(o2[:, S:], (1, 3))

    pnorm = jnp.sqrt(jnp.sum(p * p, axis=1, keepdims=True))
    pos_n = jnp.where(pnorm != 0.0, p / jnp.where(pnorm == 0.0, 1.0, pnorm), 0.0)
    s1, v1 = _norm_sv(s, vf, batch, ln_g[1], ln_b[1])
    t1 = jnp.concatenate([s1, v1, p, pos_n, jnp.zeros((N, 10), jnp.float32)],
                         axis=1)
    wts = _split_weights(Wm1[1], bm1[1], Wm2[1], bm2[1])
    aggs = []
    e2s = []
    for h, (lo, hi) in enumerate(halves):
        g2 = _sc_gather_call(t1, idxh[h])
        scat, e2_h = _edge_pass(g2, e1ts[h], ogt[:, lo:hi], None, *wts,
                                True, hi - lo)
        aggs.append(jax.ops.segment_sum(scat, tgt[lo:hi], N))
        e2s.append(e2_h)
    e2 = jnp.concatenate(e2s, axis=0)
    agg = (aggs[0] + aggs[1]) * inv_cnt[:, None]
    s = s1 + agg[:, :S]
    vf = v1 + agg[:, S:S + 48]
    p = p + agg[:, S + 48:S + 51]

    s, vf = _norm_sv(s, vf, batch, out_g, out_b)
    return (s, vf.reshape(N, 3, V), e2, p)
